# baseline (speedup 1.0000x reference)
import functools
import math

import jax
import jax.numpy as jnp
from jax import lax
from jax.experimental import pallas as pl
from jax.experimental.pallas import tpu as pltpu
from jax.experimental.pallas import tpu_sc as plsc

D_MODEL = 128
SCALE = math.sqrt(float(D_MODEL))

_NC = 2
_NS = 16
_NW = _NC * _NS
_G = 128
_GPB = 2
_C = _G * _GPB
_NBUF = 3


def _maybe(pred, fn):
    if isinstance(pred, bool):
        if pred:
            fn()
    else:
        pl.when(pred)(fn)


def _scale_rows(buf, n_rows):
    def srow(r, carry):
        r4 = r * 4
        for rr in range(4):
            for c in range(D_MODEL // 16):
                sl = pl.ds(c * 16, 16)
                buf[r4 + rr, sl] = buf[r4 + rr, sl] * SCALE
        return carry
    lax.fori_loop(0, n_rows // 4, srow, 0)


def _make_lookup(n_rows: int):
    assert n_rows % (_NW * _C) == 0
    per_w = n_rows // _NW
    n_groups = per_w // _G
    n_big = per_w // _C
    n_rounds = (n_big - 1) // _NBUF
    mesh = plsc.VectorSubcoreMesh(core_axis_name="c", subcore_axis_name="s")

    @functools.partial(
        pl.kernel,
        out_type=jax.ShapeDtypeStruct((n_rows, D_MODEL), jnp.float32),
        mesh=mesh,
        scratch_types=[
            pltpu.VMEM((n_groups, _G), jnp.int32),
            pltpu.VMEM((_NBUF, _C, D_MODEL), jnp.float32),
            pltpu.SemaphoreType.DMA((_NBUF,)),
            pltpu.SemaphoreType.DMA((_NBUF,)),
        ],
    )
    def lookup(table_hbm, idx_hbm, out_hbm, idx_v, rows_v, gsem, osem):
        wid = lax.axis_index("s") * _NC + lax.axis_index("c")
        wbase = wid * per_w
        pltpu.sync_copy(idx_hbm.at[wid], idx_v)

        def fire_gathers(s, b):
            for j in range(_GPB):
                pltpu.async_copy(
                    table_hbm.at[idx_v.at[s * _GPB + j]],
                    rows_v.at[b, pl.ds(j * _G, _G)],
                    gsem.at[b],
                )

        def wait_gathers(b):
            for j in range(_GPB):
                pltpu.make_async_copy(
                    table_hbm.at[idx_v.at[0]],
                    rows_v.at[b, pl.ds(j * _G, _G)],
                    gsem.at[b],
                ).wait()

        def wait_out(b):
            pltpu.make_async_copy(
                rows_v.at[b], out_hbm.at[pl.ds(wbase, _C)], osem.at[b]
            ).wait()

        def process(s, b):
            wait_gathers(b)
            _scale_rows(rows_v.at[b], _C)
            pltpu.async_copy(
                rows_v.at[b], out_hbm.at[pl.ds(wbase + s * _C, _C)], osem.at[b]
            )
            nb = (b - 1) % _NBUF
            ns = s + _NBUF - 1

            def _issue():
                def _wait_free():
                    wait_out(nb)

                _maybe(s > 0, _wait_free)
                fire_gathers(ns, nb)

            _maybe(ns < n_big, _issue)

        for b in range(_NBUF - 1):
            fire_gathers(b, b)

        def round_body(i, carry):
            for b in range(_NBUF):
                process(i * _NBUF + b, b)
            return carry

        lax.fori_loop(0, n_rounds, round_body, 0)
        for s in range(n_rounds * _NBUF, n_big):
            process(s, s % _NBUF)
        for b in range(_NBUF):
            wait_out(b)

    return lookup


def kernel(x, table):
    nb, nt = x.shape
    xt = x.T.reshape(_NW, -1, _G)
    out2d = _make_lookup(x.size)(table, xt)
    return out2d.reshape(nt, nb, D_MODEL).transpose(1, 0, 2)


# speedup vs baseline: 1.0110x; 1.0110x over previous
import functools
import math

import jax
import jax.numpy as jnp
from jax import lax
from jax.experimental import pallas as pl
from jax.experimental.pallas import tpu as pltpu
from jax.experimental.pallas import tpu_sc as plsc

D_MODEL = 128
SCALE = math.sqrt(float(D_MODEL))

_NC = 2
_NS = 16
_NW = _NC * _NS
_G = 128
_NBUF = 5


def _scale_rows(buf):
    def srow(r, carry):
        r4 = r * 4
        for rr in range(4):
            for c in range(D_MODEL // 16):
                sl = pl.ds(c * 16, 16)
                buf[r4 + rr, sl] = buf[r4 + rr, sl] * SCALE
        return carry
    lax.fori_loop(0, _G // 4, srow, 0)


def _make_lookup(n_rows: int):
    assert n_rows % (_NW * _G) == 0
    per_w = n_rows // _NW
    n_groups = per_w // _G
    assert n_groups % _NBUF == 0
    n_rounds = n_groups // _NBUF
    mesh = plsc.VectorSubcoreMesh(core_axis_name="c", subcore_axis_name="s")

    @functools.partial(
        pl.kernel,
        out_type=jax.ShapeDtypeStruct((n_rows, D_MODEL), jnp.float32),
        mesh=mesh,
        scratch_types=[
            pltpu.VMEM((n_groups, _G), jnp.int32),
            pltpu.VMEM((_NBUF, _G, D_MODEL), jnp.float32),
            pltpu.SemaphoreType.DMA((_NBUF,)),
            pltpu.SemaphoreType.DMA((_NBUF,)),
        ],
    )
    def lookup(table_hbm, idx_hbm, out_hbm, idx_v, rows_v, gsem, osem):
        wid = lax.axis_index("s") * _NC + lax.axis_index("c")
        wbase = wid * per_w
        pltpu.sync_copy(idx_hbm.at[wid, pl.ds(0, 8)], idx_v.at[pl.ds(0, 8)])

        for b in range(_NBUF - 1):
            pltpu.async_copy(table_hbm.at[idx_v.at[b]], rows_v.at[b], gsem.at[b])
        pltpu.sync_copy(
            idx_hbm.at[wid, pl.ds(8, n_groups - 8)],
            idx_v.at[pl.ds(8, n_groups - 8)],
        )

        def round_body(i, carry):
            gbase = i * _NBUF
            for b in range(_NBUF):
                g = gbase + b
                pltpu.make_async_copy(
                    table_hbm.at[idx_v.at[0]], rows_v.at[b], gsem.at[b]
                ).wait()
                _scale_rows(rows_v.at[b])
                pltpu.async_copy(
                    rows_v.at[b], out_hbm.at[pl.ds(wbase + g * _G, _G)], osem.at[b]
                )
                nb = (b - 1) % _NBUF
                ng = g + _NBUF - 1

                @pl.when(ng < n_groups)
                def _issue(nb=nb, ng=ng, g=g):
                    @pl.when(g > 0)
                    def _wait_free():
                        pltpu.make_async_copy(
                            rows_v.at[nb], out_hbm.at[pl.ds(wbase, _G)], osem.at[nb]
                        ).wait()

                    pltpu.async_copy(
                        table_hbm.at[idx_v.at[ng]], rows_v.at[nb], gsem.at[nb]
                    )
            return carry

        lax.fori_loop(0, n_rounds, round_body, 0)
        for b in range(_NBUF):
            pltpu.make_async_copy(
                rows_v.at[b], out_hbm.at[pl.ds(wbase, _G)], osem.at[b]
            ).wait()

    return lookup


def kernel(x, table):
    nb, nt = x.shape
    xt = x.T.reshape(_NW, -1, _G)
    out2d = _make_lookup(x.size)(table, xt)
    return out2d.reshape(nt, nb, D_MODEL).transpose(1, 0, 2)


# speedup vs baseline: 1.0191x; 1.0080x over previous
import functools
import math

import jax
import jax.numpy as jnp
from jax import lax
from jax.experimental import pallas as pl
from jax.experimental.pallas import tpu as pltpu
from jax.experimental.pallas import tpu_sc as plsc

D_MODEL = 128
SCALE = math.sqrt(float(D_MODEL))

_NC = 2
_NS = 16
_NW = _NC * _NS
_G = 128
_NBUF = 5


def _scale_rows(buf):
    def srow(r, carry):
        for c in range(D_MODEL // 16):
            sl = pl.ds(c * 16, 16)
            buf[r, sl] = buf[r, sl] * SCALE
        return carry
    lax.fori_loop(0, _G, srow, 0)


def _make_lookup(n_rows: int):
    assert n_rows % (_NW * _G) == 0
    per_w = n_rows // _NW
    n_groups = per_w // _G
    assert n_groups % _NBUF == 0
    n_rounds = n_groups // _NBUF
    mesh = plsc.VectorSubcoreMesh(core_axis_name="c", subcore_axis_name="s")

    @functools.partial(
        pl.kernel,
        out_type=jax.ShapeDtypeStruct((n_rows, D_MODEL), jnp.float32),
        mesh=mesh,
        scratch_types=[
            pltpu.VMEM((n_groups, _G), jnp.int32),
            pltpu.VMEM((_NBUF, _G, D_MODEL), jnp.float32),
            pltpu.SemaphoreType.DMA((_NBUF,)),
            pltpu.SemaphoreType.DMA((_NBUF,)),
        ],
    )
    def lookup(table_hbm, idx_hbm, out_hbm, idx_v, rows_v, gsem, osem):
        wid = lax.axis_index("s") * _NC + lax.axis_index("c")
        wbase = wid * per_w
        pltpu.sync_copy(idx_hbm.at[wid, pl.ds(0, 8)], idx_v.at[pl.ds(0, 8)])

        for b in range(_NBUF - 1):
            pltpu.async_copy(table_hbm.at[idx_v.at[b]], rows_v.at[b], gsem.at[b])
        pltpu.sync_copy(
            idx_hbm.at[wid, pl.ds(8, n_groups - 8)],
            idx_v.at[pl.ds(8, n_groups - 8)],
        )

        def round_body(i, carry):
            gbase = i * _NBUF
            for b in range(_NBUF):
                g = gbase + b
                pltpu.make_async_copy(
                    table_hbm.at[idx_v.at[0]], rows_v.at[b], gsem.at[b]
                ).wait()
                _scale_rows(rows_v.at[b])
                pltpu.async_copy(
                    rows_v.at[b], out_hbm.at[pl.ds(wbase + g * _G, _G)], osem.at[b]
                )
                nb = (b - 1) % _NBUF
                ng = g + _NBUF - 1

                @pl.when(ng < n_groups)
                def _issue(nb=nb, ng=ng, g=g):
                    @pl.when(g > 0)
                    def _wait_free():
                        pltpu.make_async_copy(
                            rows_v.at[nb], out_hbm.at[pl.ds(wbase, _G)], osem.at[nb]
                        ).wait()

                    pltpu.async_copy(
                        table_hbm.at[idx_v.at[ng]], rows_v.at[nb], gsem.at[nb]
                    )
            return carry

        lax.fori_loop(0, n_rounds, round_body, 0)
        for b in range(_NBUF):
            pltpu.make_async_copy(
                rows_v.at[b], out_hbm.at[pl.ds(wbase, _G)], osem.at[b]
            ).wait()

    return lookup


def kernel(x, table):
    nb, nt = x.shape
    xt = x.T.reshape(_NW, -1, _G)
    out2d = _make_lookup(x.size)(table, xt)
    return out2d.reshape(nt, nb, D_MODEL).transpose(1, 0, 2)
